# baseline (speedup 1.0000x reference)
import jax
import jax.numpy as jnp
from jax import lax
from jax.experimental import pallas as pl
from jax.experimental.pallas import tpu as pltpu
from jax.experimental.pallas import tpu_sc as plsc

_B, _S, _D = 4, 2048, 1024
_NW = 32
_SPW = _S // _NW
_SCH = 4
_NCH = _SPW // _SCH


def _bcast16(vec, lane):
    return lax.gather(
        vec, jnp.full((16, 1), lane, jnp.int32),
        lax.GatherDimensionNumbers(offset_dims=(),
                                   collapsed_slice_dims=(0,),
                                   start_index_map=(0,)),
        slice_sizes=(1,),
        mode=lax.GatherScatterMode.PROMISE_IN_BOUNDS)


def _sc_body(x_hbm, ids_hbm, seg_hbm, pos_hbm, out_hbm,
             xbuf, obuf, pbuf, idbuf, idibuf, segbuf, dsbuf, insem, outsem):
    cid = lax.axis_index("c")
    sid = lax.axis_index("s")
    wid = sid * 2 + cid
    s_base = wid * _SPW

    for b in range(_B):
        pltpu.sync_copy(ids_hbm.at[b, pl.ds(s_base, _SPW)], idibuf.at[b])
    pltpu.sync_copy(seg_hbm, segbuf)
    for b in range(_B):
        for i in range(_SPW // 16):
            sl = pl.ds(i * 16, 16)
            idbuf[b, sl] = idibuf[b, sl].astype(jnp.float32)

    @plsc.parallel_loop(0, _D // 16, 1, unroll=4)
    def _(i):
        sl = pl.ds(i * 16, 16)
        dsbuf[sl] = segbuf[1, sl] - segbuf[0, sl]

    def start_in(c):
        par = lax.rem(c, 3)
        s0 = s_base + c * _SCH
        pltpu.async_copy(x_hbm.at[:, pl.ds(s0, _SCH), :], xbuf.at[par], insem)
        pltpu.async_copy(pos_hbm.at[pl.ds(s0, _SCH), :], pbuf.at[par], insem)

    def wait_in():
        pltpu.make_async_copy(x_hbm.at[:, pl.ds(0, _SCH), :], xbuf.at[0],
                              insem).wait()
        pltpu.make_async_copy(pos_hbm.at[pl.ds(0, _SCH), :], pbuf.at[0],
                              insem).wait()

    def start_out(c):
        par = lax.rem(c, 3)
        s0 = s_base + c * _SCH
        pltpu.async_copy(obuf.at[par], out_hbm.at[:, pl.ds(s0, _SCH), :],
                         outsem)


    def wait_out():
        pltpu.make_async_copy(obuf.at[0], out_hbm.at[:, pl.ds(0, _SCH), :],
                              outsem).wait()

    start_in(0)
    start_in(1)

    def chunk_body(c, _):
        par = lax.rem(c, 3)
        opar = lax.rem(c, 3)

        @pl.when(c >= 3)
        def _():
            wait_out()

        @pl.when(c + 2 < _NCH)
        def _():
            start_in(c + 2)

        wait_in()

        win = (c // _SCH) * 16
        lane0 = lax.rem(c, _SCH) * _SCH
        idvecs = [idbuf[b, pl.ds(win, 16)] for b in range(_B)]

        @plsc.parallel_loop(0, _SCH * (_D // 16), 1, unroll=8)
        def _(i):
            s = i // (_D // 16)
            col = lax.rem(i, _D // 16) * 16
            sl = pl.ds(col, 16)
            t = pbuf[par, s, sl] + segbuf[0, sl]
            dvv = dsbuf[sl]
            for b in range(_B):
                m = _bcast16(idvecs[b], lane0 + s)
                obuf[opar, b, s, sl] = (xbuf[par, b, s, sl] + t) + m * dvv

        start_out(c)
        return 0
    lax.fori_loop(0, _NCH, chunk_body, 0)
    wait_out()
    wait_out()
    wait_out()


@jax.jit
def _sc_call(x, ids, seg, pos):
    mesh = plsc.VectorSubcoreMesh(core_axis_name="c", subcore_axis_name="s")
    return pl.kernel(
        _sc_body,
        out_type=jax.ShapeDtypeStruct((_B, _S, _D), jnp.float32),
        mesh=mesh,
        scratch_types=[
            pltpu.VMEM((3, _B, _SCH, _D), jnp.float32),
            pltpu.VMEM((3, _B, _SCH, _D), jnp.float32),
            pltpu.VMEM((3, _SCH, _D), jnp.float32),
            pltpu.VMEM((_B, _SPW), jnp.float32),
            pltpu.VMEM((_B, _SPW), jnp.int32),
            pltpu.VMEM((2, _D), jnp.float32),
            pltpu.VMEM((_D,), jnp.float32),
            pltpu.SemaphoreType.DMA,
            pltpu.SemaphoreType.DMA,
        ],
    )(x, ids, seg, pos)


def kernel(x, segment_input_ids, segment_table, position_table):
    return _sc_call(x, segment_input_ids, segment_table, position_table)


# speedup vs baseline: 1.0004x; 1.0004x over previous
import jax
import jax.numpy as jnp
from jax import lax
from jax.experimental import pallas as pl
from jax.experimental.pallas import tpu as pltpu
from jax.experimental.pallas import tpu_sc as plsc

_B, _S, _D = 4, 2048, 1024
_NW = 32
_SPW = _S // _NW
_SCH = 4
_NCH = _SPW // _SCH


def _bcast16(vec, lane):
    return lax.gather(
        vec, jnp.full((16, 1), lane, jnp.int32),
        lax.GatherDimensionNumbers(offset_dims=(),
                                   collapsed_slice_dims=(0,),
                                   start_index_map=(0,)),
        slice_sizes=(1,),
        mode=lax.GatherScatterMode.PROMISE_IN_BOUNDS)


def _sc_body(x_hbm, ids_hbm, seg_hbm, pos_hbm, out_hbm,
             xbuf, obuf, pbuf, idbuf, idibuf, segbuf, dsbuf, insem, outsem):
    cid = lax.axis_index("c")
    sid = lax.axis_index("s")
    wid = sid * 2 + cid
    s_base = wid * _SPW

    for b in range(_B):
        pltpu.sync_copy(ids_hbm.at[b, pl.ds(s_base, _SPW)], idibuf.at[b])
    pltpu.sync_copy(seg_hbm, segbuf)
    for b in range(_B):
        for i in range(_SPW // 16):
            sl = pl.ds(i * 16, 16)
            idbuf[b, sl] = idibuf[b, sl].astype(jnp.float32)

    @plsc.parallel_loop(0, _D // 16, 1, unroll=4)
    def _(i):
        sl = pl.ds(i * 16, 16)
        dsbuf[sl] = segbuf[1, sl] - segbuf[0, sl]

    def start_in(c):
        par = lax.rem(c, 3)
        s0 = s_base + c * _SCH
        pltpu.async_copy(x_hbm.at[:, pl.ds(s0, _SCH), :], xbuf.at[par], insem)
        pltpu.async_copy(pos_hbm.at[pl.ds(s0, _SCH), :], pbuf.at[par], insem)

    def wait_in():
        pltpu.make_async_copy(x_hbm.at[:, pl.ds(0, _SCH), :], xbuf.at[0],
                              insem).wait()
        pltpu.make_async_copy(pos_hbm.at[pl.ds(0, _SCH), :], pbuf.at[0],
                              insem).wait()

    def start_out(c):
        par = lax.rem(c, 2)
        s0 = s_base + c * _SCH
        pltpu.async_copy(obuf.at[par], out_hbm.at[:, pl.ds(s0, _SCH), :],
                         outsem)


    def wait_out():
        pltpu.make_async_copy(obuf.at[0], out_hbm.at[:, pl.ds(0, _SCH), :],
                              outsem).wait()

    start_in(0)
    start_in(1)

    def chunk_body(c, _):
        par = lax.rem(c, 3)
        opar = lax.rem(c, 2)

        @pl.when(c + 2 < _NCH)
        def _():
            start_in(c + 2)

        @pl.when(c >= 2)
        def _():
            wait_out()

        wait_in()

        win = (c // _SCH) * 16
        lane0 = lax.rem(c, _SCH) * _SCH
        idvecs = [idbuf[b, pl.ds(win, 16)] for b in range(_B)]

        @plsc.parallel_loop(0, _SCH * (_D // 16), 1, unroll=8)
        def _(i):
            s = i // (_D // 16)
            col = lax.rem(i, _D // 16) * 16
            sl = pl.ds(col, 16)
            t = pbuf[par, s, sl] + segbuf[0, sl]
            dvv = dsbuf[sl]
            for b in range(_B):
                m = _bcast16(idvecs[b], lane0 + s)
                obuf[opar, b, s, sl] = (xbuf[par, b, s, sl] + t) + m * dvv

        start_out(c)
        return 0
    lax.fori_loop(0, _NCH, chunk_body, 0)
    wait_out()
    wait_out()


@jax.jit
def _sc_call(x, ids, seg, pos):
    mesh = plsc.VectorSubcoreMesh(core_axis_name="c", subcore_axis_name="s")
    return pl.kernel(
        _sc_body,
        out_type=jax.ShapeDtypeStruct((_B, _S, _D), jnp.float32),
        mesh=mesh,
        scratch_types=[
            pltpu.VMEM((3, _B, _SCH, _D), jnp.float32),
            pltpu.VMEM((2, _B, _SCH, _D), jnp.float32),
            pltpu.VMEM((3, _SCH, _D), jnp.float32),
            pltpu.VMEM((_B, _SPW), jnp.float32),
            pltpu.VMEM((_B, _SPW), jnp.int32),
            pltpu.VMEM((2, _D), jnp.float32),
            pltpu.VMEM((_D,), jnp.float32),
            pltpu.SemaphoreType.DMA,
            pltpu.SemaphoreType.DMA,
        ],
    )(x, ids, seg, pos)


def kernel(x, segment_input_ids, segment_table, position_table):
    return _sc_call(x, segment_input_ids, segment_table, position_table)
